# baseline (speedup 1.0000x reference)
import jax
import jax.numpy as jnp
from jax.experimental import pallas as pl
from jax.experimental.pallas import tpu as pltpu

ALPHA_C = 0.1
K_NN = 4
RADIUS_C = 0.07
H2 = 0.03 ** 2
EPS_C = 1e-12

B, C, N = 16, 3, 2048
C8 = 8
K40 = 40
ROWS = 512
NBLK = N // ROWS


def _loss_kernel(gt_row_ref, pred_col_ref,
                 lhs_g_ref, lhs_qn_ref, lhs_qe_ref, rhs_ref,
                 rinv_ref, out_ref, colmin_ref):
    b = pl.program_id(0)
    i = pl.program_id(1)

    @pl.when((b == 0) & (i == 0))
    def _init_out():
        out_ref[...] = jnp.zeros((1, 1), jnp.float32)

    @pl.when(i == 0)
    def _init_colmin():
        colmin_ref[...] = jnp.full((1, N), jnp.inf, jnp.float32)

    lhs = jnp.concatenate([lhs_g_ref[0], lhs_qn_ref[0], lhs_qe_ref[0]],
                          axis=0)
    ab = jnp.dot(lhs, rhs_ref[0],
                 preferred_element_type=jnp.float32)

    g = gt_row_ref[0]
    p = pred_col_ref[0]
    g2 = 0.25 * jnp.sum(g * g, axis=1, keepdims=True)
    p2 = jnp.sum(p * p, axis=0, keepdims=True)
    d_n = (g2 + p2) + ab[:ROWS]
    dpp_n = ab[ROWS:2 * ROWS]

    dpp_nb = dpp_n.astype(jnp.bfloat16)
    delta = (ab[2 * ROWS:] - dpp_nb.astype(jnp.float32)
             ).astype(jnp.bfloat16)

    rinv = rinv_ref[0, 0, 0]
    inv_bn = 1.0 / (B * N)

    rowmin = jnp.min(d_n, axis=1)
    colmin_ref[...] = jnp.minimum(colmin_ref[...],
                                  jnp.min(d_n, axis=0, keepdims=True))
    acc = (0.8 * inv_bn) * rinv * jnp.sum(rowmin)

    inf_b = jnp.array(jnp.inf, jnp.bfloat16)
    m = jnp.min(dpp_nb, axis=1, keepdims=True)
    dpp_nb = jnp.where(dpp_nb == m, inf_b, dpp_nb)
    rep = jnp.zeros((), jnp.float32)
    for _ in range(K_NN):
        m = jnp.min(dpp_nb, axis=1, keepdims=True)
        sel = dpp_nb == m
        db = jnp.min(jnp.where(sel, delta, inf_b), axis=1, keepdims=True)
        dpp_nb = jnp.where(sel, inf_b, dpp_nb)
        e = m.astype(jnp.float32) + db.astype(jnp.float32)
        d2 = jnp.maximum(e, EPS_C)
        dist = jnp.sqrt(d2)
        w = jnp.exp(-d2 * (1.0 / H2))
        rep = rep + jnp.sum((RADIUS_C - dist) * w)
    acc = acc + (ALPHA_C * inv_bn / K_NN) * rep

    tail = jnp.where(i == NBLK - 1,
                     (0.2 * inv_bn) * rinv * jnp.sum(colmin_ref[...]),
                     0.0)
    out_ref[...] = out_ref[...] + (acc + tail)


def _split3(x):
    f32 = jnp.float32
    bf16 = jnp.bfloat16
    h = x.astype(bf16)
    r = x - h.astype(f32)
    mid = r.astype(bf16)
    l = (r - mid.astype(f32)).astype(bf16)
    return h, mid, l


def kernel(pred, gt, pcd_radius):
    f32 = jnp.float32
    bf16 = jnp.bfloat16
    pad_t = [(0, 0), (0, 0), (0, C8 - C)]
    gt_t = jnp.pad(jnp.transpose(-2.0 * gt, (0, 2, 1)), pad_t)
    pred_t = jnp.pad(jnp.transpose(-2.0 * pred, (0, 2, 1)), pad_t)
    pred_p = jnp.pad(pred, [(0, 0), (0, C8 - C), (0, 0)])

    n2p = jnp.sum(pred * pred, axis=1)
    n2g = jnp.sum(gt * gt, axis=1)

    gb = gt_t.astype(bf16)
    qh = pred_t.astype(bf16)
    ql = (pred_t - qh.astype(f32)).astype(bf16)
    ph = pred_p.astype(bf16)
    pl_ = (pred_p - ph.astype(f32)).astype(bf16)

    p2h, p2m, p2l = _split3(n2p)
    g2h, g2m, g2l = _split3(n2g)
    q2h, q2m, q2l = p2h, p2m, p2l

    ones_r = jnp.ones((B, N, 3), bf16)
    zero8 = jnp.zeros((B, N, C8), bf16)
    zero2 = jnp.zeros((B, N, 2), bf16)

    def stk(*xs):
        return jnp.stack(xs, axis=2)

    zero3 = jnp.zeros((B, N, 3), bf16)
    lhs_g = jnp.concatenate(
        [gb, zero8, zero8, zero8, zero3, zero3, zero2], axis=2)
    lhs_qn = jnp.concatenate(
        [qh, zero8, zero8, zero8, ones_r, stk(q2h, q2m, q2l), zero2], axis=2)
    lhs_qe = jnp.concatenate(
        [qh, ql, qh, ql, ones_r, stk(q2h, q2m, q2l), zero2], axis=2)

    ones_c = jnp.ones((B, 3, N), bf16)
    zero2c = jnp.zeros((B, 2, N), bf16)
    p2rows = jnp.stack([p2h, p2m, p2l], axis=1)
    rhs = jnp.concatenate([ph, pl_, pl_, ph, p2rows, ones_c, zero2c], axis=1)

    rinv = (1.0 / pcd_radius).reshape(B, 1, 1)
    row_spec = pl.BlockSpec((1, ROWS, C8), lambda b, i: (b, i, 0))
    col_spec = pl.BlockSpec((1, C8, N), lambda b, i: (b, 0, 0))
    lhs_spec = pl.BlockSpec((1, ROWS, K40), lambda b, i: (b, i, 0))
    rhs_spec = pl.BlockSpec((1, K40, N), lambda b, i: (b, 0, 0))
    out = pl.pallas_call(
        _loss_kernel,
        grid=(B, NBLK),
        in_specs=[
            row_spec, col_spec,
            lhs_spec, lhs_spec, lhs_spec, rhs_spec,
            pl.BlockSpec((1, 1, 1), lambda b, i: (b, 0, 0)),
        ],
        out_specs=pl.BlockSpec((1, 1), lambda b, i: (0, 0)),
        out_shape=jax.ShapeDtypeStruct((1, 1), jnp.float32),
        scratch_shapes=[pltpu.VMEM((1, N), jnp.float32)],
    )(gt_t, pred_p, lhs_g, lhs_qn, lhs_qe, rhs, rinv)
    return out[0, 0]
